# baseline (speedup 1.0000x reference)
import jax
import jax.numpy as jnp
from jax import lax
from jax.experimental import pallas as pl
from jax.experimental.pallas import tpu as pltpu
from jax.experimental.pallas import tpu_sc as plsc

MODEL_DIM = 64
NUM_WORKERS = 32
LANES = 16
NBUF = 2
TR_PITCH = 136


def kernel(x, table):
    batch, hist = x.shape

    b_w = batch // NUM_WORKERS
    n_groups = b_w // LANES

    mesh = plsc.VectorSubcoreMesh(core_axis_name="core",
                                  subcore_axis_name="subcore")

    @pl.kernel(out_type=jax.ShapeDtypeStruct(
                   (hist, MODEL_DIM // 8, NUM_WORKERS, 8, 128),
                   table.dtype),
               mesh=mesh,
               scratch_types=[
                   pltpu.VMEM((b_w, hist), jnp.int32),
                   pltpu.VMEM((hist, b_w), jnp.int32),
                   pltpu.VMEM((NBUF, b_w, MODEL_DIM), jnp.float32),
                   pltpu.VMEM((NBUF, MODEL_DIM, TR_PITCH), jnp.float32),
                   pltpu.SemaphoreType.DMA((NBUF,)),
                   pltpu.SemaphoreType.DMA((NBUF,)),
                   pltpu.SemaphoreType.DMA,
               ],
               compiler_params=pltpu.CompilerParams(use_tc_tiling_on_sc=False,
                                                    needs_layout_passes=False))
    def gather_kernel(table_hbm, x_hbm, out_hbm,
                      xraw_v, idx_v, rows_v, tr_v, gsem, ssem, isem):
        wid = lax.axis_index("subcore") * 2 + lax.axis_index("core")
        pltpu.async_copy(x_hbm.at[pl.ds(wid * b_w, b_w)],
                         xraw_v, isem).wait()

        iota = lax.iota(jnp.int32, LANES)
        row_vecs = [iota + g * LANES for g in range(n_groups)]

        @pl.loop(0, hist, step=8)
        def _(t0):
            tcols = [jnp.full((LANES,), t0 + k, jnp.int32) for k in range(8)]
            for g in range(n_groups):
                vs = [plsc.load_gather(xraw_v, [row_vecs[g], tcols[k]])
                      for k in range(8)]
                for k in range(8):
                    idx_v[t0 + k, pl.ds(g * LANES, LANES)] = vs[k]

        def start_gather(t, buf):
            pltpu.make_async_copy(table_hbm.at[idx_v.at[t]],
                                  rows_v.at[buf], gsem.at[buf]).start()

        def wait_gather(t, buf):
            pltpu.make_async_copy(table_hbm.at[idx_v.at[t]],
                                  rows_v.at[buf], gsem.at[buf]).wait()

        d_vecs = [iota + d0 for d0 in range(0, MODEL_DIM, LANES)]

        def transpose(buf):
            src = rows_v.at[buf]
            dst = tr_v.at[buf]

            @pl.loop(0, b_w, step=8)
            def _(b0):
                for bb in range(8):
                    col = jnp.full((LANES,), b0 + bb, jnp.int32)
                    vs = [src[b0 + bb, pl.ds(j * LANES, LANES)]
                          for j in range(MODEL_DIM // LANES)]
                    for j in range(MODEL_DIM // LANES):
                        plsc.store_scatter(dst, [d_vecs[j], col], vs[j])

        def start_store(t, buf):
            for dt in range(MODEL_DIM // 8):
                pltpu.make_async_copy(
                    tr_v.at[buf, pl.ds(dt * 8, 8), pl.ds(0, b_w)],
                    out_hbm.at[t, dt, wid], ssem.at[buf]).start()

        def wait_store(t, buf):
            for dt in range(MODEL_DIM // 8):
                pltpu.make_async_copy(
                    tr_v.at[buf, pl.ds(dt * 8, 8), pl.ds(0, b_w)],
                    out_hbm.at[t, dt, wid], ssem.at[buf]).wait()

        for buf in range(NBUF):
            start_gather(buf, buf)

        @pl.loop(0, hist - NBUF, step=NBUF)
        def _(t0):
            for buf in range(NBUF):
                t = t0 + buf
                wait_gather(t, buf)

                @pl.when(t0 >= NBUF)
                def _():
                    wait_store(t - NBUF, buf)

                transpose(buf)
                start_store(t, buf)
                start_gather(t + NBUF, buf)

        for buf in range(NBUF):
            t = hist - NBUF + buf
            wait_gather(t, buf)
            wait_store(t - NBUF, buf)
            transpose(buf)
            start_store(t, buf)
        for buf in range(NBUF):
            wait_store(hist - NBUF + buf, buf)

    out5 = gather_kernel(table, x.astype(jnp.int32))
    return out5.transpose(2, 4, 0, 1, 3).reshape(batch, hist, MODEL_DIM)


# speedup vs baseline: 1.1689x; 1.1689x over previous
import jax
import jax.numpy as jnp
from jax import lax
from jax.experimental import pallas as pl
from jax.experimental.pallas import tpu as pltpu
from jax.experimental.pallas import tpu_sc as plsc

MODEL_DIM = 64
NUM_WORKERS = 32
LANES = 16
NBUF = 4
PITCH = 72


def kernel(x, table):
    batch, hist = x.shape
    table_p = jnp.pad(table, ((0, 0), (0, PITCH - MODEL_DIM)))

    b_w = batch // NUM_WORKERS
    n_groups = b_w // LANES

    mesh = plsc.VectorSubcoreMesh(core_axis_name="core",
                                  subcore_axis_name="subcore")

    @pl.kernel(out_type=jax.ShapeDtypeStruct(
                   (hist, MODEL_DIM // 8, NUM_WORKERS, 8, 128),
                   table.dtype),
               mesh=mesh,
               scratch_types=[
                   pltpu.VMEM((b_w, hist), jnp.int32),
                   pltpu.VMEM((hist, b_w), jnp.int32),
                   pltpu.VMEM((NBUF, b_w, PITCH), jnp.float32),
                   pltpu.VMEM((NBUF, MODEL_DIM, b_w), jnp.float32),
                   pltpu.SemaphoreType.DMA((NBUF,)),
                   pltpu.SemaphoreType.DMA((NBUF,)),
                   pltpu.SemaphoreType.DMA,
               ],
               compiler_params=pltpu.CompilerParams(use_tc_tiling_on_sc=False,
                                                    needs_layout_passes=False))
    def gather_kernel(table_hbm, x_hbm, out_hbm,
                      xraw_v, idx_v, rows_v, tr_v, gsem, ssem, isem):
        wid = lax.axis_index("subcore") * 2 + lax.axis_index("core")
        pltpu.async_copy(x_hbm.at[pl.ds(wid * b_w, b_w)],
                         xraw_v, isem).wait()

        iota = lax.iota(jnp.int32, LANES)
        row_vecs = [iota + g * LANES for g in range(n_groups)]

        @pl.loop(0, hist, step=8)
        def _(t0):
            tcols = [jnp.full((LANES,), t0 + k, jnp.int32) for k in range(8)]
            for g in range(n_groups):
                vs = [plsc.load_gather(xraw_v, [row_vecs[g], tcols[k]])
                      for k in range(8)]
                for k in range(8):
                    idx_v[t0 + k, pl.ds(g * LANES, LANES)] = vs[k]

        def start_gather(t, buf):
            pltpu.make_async_copy(table_hbm.at[idx_v.at[t]],
                                  rows_v.at[buf], gsem.at[buf]).start()

        def wait_gather(t, buf):
            pltpu.make_async_copy(table_hbm.at[idx_v.at[t]],
                                  rows_v.at[buf], gsem.at[buf]).wait()

        def transpose(buf):
            src = rows_v.at[buf]
            dst = tr_v.at[buf]

            @pl.loop(0, MODEL_DIM, step=8)
            def _(d0):
                cols = [jnp.full((LANES,), d0 + k, jnp.int32)
                        for k in range(8)]
                for g in range(n_groups):
                    vs = [plsc.load_gather(src, [row_vecs[g], cols[k]])
                          for k in range(8)]
                    for k in range(8):
                        dst[d0 + k, pl.ds(g * LANES, LANES)] = vs[k]

        def start_store(t, buf):
            for dt in range(MODEL_DIM // 8):
                pltpu.make_async_copy(tr_v.at[buf, pl.ds(dt * 8, 8)],
                                      out_hbm.at[t, dt, wid],
                                      ssem.at[buf]).start()

        def wait_store(t, buf):
            for dt in range(MODEL_DIM // 8):
                pltpu.make_async_copy(tr_v.at[buf, pl.ds(dt * 8, 8)],
                                      out_hbm.at[t, dt, wid],
                                      ssem.at[buf]).wait()

        for buf in range(NBUF):
            start_gather(buf, buf)

        @pl.loop(0, hist - NBUF, step=NBUF)
        def _(t0):
            for buf in range(NBUF):
                t = t0 + buf
                wait_gather(t, buf)

                @pl.when(t0 >= NBUF)
                def _():
                    wait_store(t - NBUF, buf)

                transpose(buf)
                start_store(t, buf)
                start_gather(t + NBUF, buf)

        for buf in range(NBUF):
            t = hist - NBUF + buf
            wait_gather(t, buf)
            wait_store(t - NBUF, buf)
            transpose(buf)
            start_store(t, buf)
        for buf in range(NBUF):
            wait_store(hist - NBUF + buf, buf)

    out5 = gather_kernel(table_p, x.astype(jnp.int32))
    return out5.transpose(2, 4, 0, 1, 3).reshape(batch, hist, MODEL_DIM)
